# baseline (speedup 1.0000x reference)
import jax
import jax.numpy as jnp
from jax.experimental import pallas as pl
from jax.experimental.pallas import tpu as pltpu

_L = 16
_BB = 1024

_DN_TL = (((0,), (1,)), ((), ()))
_DN_RR = (((1,), (1,)), ((), ()))


def _prnn_block(x_ref, w2_ref, b_ref, out_ref):
    xb = x_ref[:, :32]
    rows = ([i * 128 + 32 + i for i in range(_L)]
            + [i * 128 + 48 + i for i in range(_L)])
    wsel = jnp.concatenate([w2_ref[ri:ri + 1, :] for ri in rows], axis=0)
    wxsel = wsel[:, :32]
    wr = wsel[:, 32:]
    w15 = w2_ref[(_L - 1) * 128:_L * 128, :]
    acc = jax.lax.dot_general(wxsel, xb, _DN_RR,
                              preferred_element_type=jnp.float32)
    vs0 = []
    vs1 = []
    for idx in range(_L - 1):
        b0 = b_ref[idx:idx + 1, 32 + idx:33 + idx]
        b1 = b_ref[idx:idx + 1, 48 + idx:49 + idx]
        v0 = jnp.maximum(acc[idx:idx + 1, :] + b0, 0.0)
        v1 = jnp.maximum(acc[16 + idx:17 + idx, :] + b1, 0.0)
        acc = acc + wr[:, idx:idx + 1] * v0 + wr[:, 16 + idx:17 + idx] * v1
        vs0.append(v0)
        vs1.append(v1)
    zero = jnp.zeros_like(vs0[0])
    a_t = jnp.concatenate(vs0 + [zero] + vs1 + [zero], axis=0)
    p15 = jnp.dot(xb, w15[:, :32].T,
                  preferred_element_type=jnp.float32)
    rec15 = jax.lax.dot_general(a_t, w15[:, 32:], _DN_TL,
                                preferred_element_type=jnp.float32)
    out_ref[...] = jnp.maximum(p15 + rec15 + b_ref[_L - 1:_L, :], 0.0)


def kernel(x, W, b):
    batch = x.shape[0]
    w2 = W.reshape(_L * 128, 64)
    grid = (batch // _BB,)
    return pl.pallas_call(
        _prnn_block,
        grid=grid,
        in_specs=[
            pl.BlockSpec((_BB, 128), lambda i: (i, 0)),
            pl.BlockSpec((_L * 128, 64), lambda i: (0, 0)),
            pl.BlockSpec((_L, 128), lambda i: (0, 0)),
        ],
        out_specs=pl.BlockSpec((_BB, 128), lambda i: (i, 0)),
        out_shape=jax.ShapeDtypeStruct((batch, 128), jnp.float32),
        compiler_params=pltpu.CompilerParams(
            dimension_semantics=("parallel",)),
    )(x, w2, b)


# speedup vs baseline: 1.1291x; 1.1291x over previous
import jax
import jax.numpy as jnp
from jax.experimental import pallas as pl
from jax.experimental.pallas import tpu as pltpu

_L = 16
_BB = 2048

_DN_TL = (((0,), (1,)), ((), ()))
_DN_RR = (((1,), (1,)), ((), ()))


def _prnn_block(x_ref, w2_ref, b_ref, out_ref):
    xb = x_ref[:, :32]
    rows = ([i * 128 + 32 + i for i in range(_L)]
            + [i * 128 + 48 + i for i in range(_L)])
    wsel = jnp.concatenate([w2_ref[ri:ri + 1, :] for ri in rows], axis=0)
    wxsel = wsel[:, :32]
    wr = wsel[:, 32:]
    w15 = w2_ref[(_L - 1) * 128:_L * 128, :]
    acc = jax.lax.dot_general(wxsel, xb, _DN_RR,
                              preferred_element_type=jnp.float32)
    vs0 = []
    vs1 = []
    for idx in range(_L - 1):
        b0 = b_ref[idx:idx + 1, 32 + idx:33 + idx]
        b1 = b_ref[idx:idx + 1, 48 + idx:49 + idx]
        v0 = jnp.maximum(acc[idx:idx + 1, :] + b0, 0.0)
        v1 = jnp.maximum(acc[16 + idx:17 + idx, :] + b1, 0.0)
        acc = acc + wr[:, idx:idx + 1] * v0 + wr[:, 16 + idx:17 + idx] * v1
        vs0.append(v0)
        vs1.append(v1)
    zero = jnp.zeros_like(vs0[0])
    a_t = jnp.concatenate(vs0 + [zero] + vs1 + [zero], axis=0)
    p15 = jnp.dot(xb, w15[:, :32].T,
                  preferred_element_type=jnp.float32)
    rec15 = jax.lax.dot_general(a_t, w15[:, 32:], _DN_TL,
                                preferred_element_type=jnp.float32)
    out_ref[...] = jnp.maximum(p15 + rec15 + b_ref[_L - 1:_L, :], 0.0)


def kernel(x, W, b):
    batch = x.shape[0]
    bb = _BB if batch % _BB == 0 else batch
    w2 = W.reshape(_L * 128, 64)
    grid = (batch // bb,)
    return pl.pallas_call(
        _prnn_block,
        grid=grid,
        in_specs=[
            pl.BlockSpec((bb, 128), lambda i: (i, 0)),
            pl.BlockSpec((_L * 128, 64), lambda i: (0, 0)),
            pl.BlockSpec((_L, 128), lambda i: (0, 0)),
        ],
        out_specs=pl.BlockSpec((bb, 128), lambda i: (i, 0)),
        out_shape=jax.ShapeDtypeStruct((batch, 128), jnp.float32),
        compiler_params=pltpu.CompilerParams(
            dimension_semantics=("parallel",)),
    )(x, w2, b)


# speedup vs baseline: 1.1525x; 1.0207x over previous
import jax
import jax.numpy as jnp
from jax.experimental import pallas as pl
from jax.experimental.pallas import tpu as pltpu

_L = 16
_BB = 2048

_DN_TL = (((0,), (1,)), ((), ()))
_DN_RR = (((1,), (1,)), ((), ()))


def _prnn_block(x_ref, w2_ref, b_ref, out_ref):
    xb = x_ref[:, :32]
    rows = ([i * 128 + 32 + i for i in range(_L)]
            + [i * 128 + 48 + i for i in range(_L)])
    wsel = jnp.concatenate([w2_ref[ri:ri + 1, :] for ri in rows], axis=0)
    wxsel = wsel[:, :32]
    wr = wsel[:, 32:]
    w15 = w2_ref[(_L - 1) * 128:_L * 128, :]
    acc = jax.lax.dot_general(wxsel, xb, _DN_RR,
                              preferred_element_type=jnp.float32)
    acc_a = jnp.concatenate([acc[0:8, :], acc[16:24, :]], axis=0)
    wr_a = jnp.concatenate([wr[0:8, :], wr[16:24, :]], axis=0)
    vs0 = []
    vs1 = []
    for idx in range(8):
        b0 = b_ref[idx:idx + 1, 32 + idx:33 + idx]
        b1 = b_ref[idx:idx + 1, 48 + idx:49 + idx]
        v0 = jnp.maximum(acc_a[idx:idx + 1, :] + b0, 0.0)
        v1 = jnp.maximum(acc_a[8 + idx:9 + idx, :] + b1, 0.0)
        acc_a = (acc_a + wr_a[:, idx:idx + 1] * v0
                 + wr_a[:, 16 + idx:17 + idx] * v1)
        vs0.append(v0)
        vs1.append(v1)
    va = jnp.concatenate(vs0 + vs1, axis=0)
    wr_b = jnp.concatenate([wr[8:16, :], wr[24:32, :]], axis=0)
    wr_bridge = jnp.concatenate([wr_b[:, 0:8], wr_b[:, 16:24]], axis=1)
    acc_b = (jnp.concatenate([acc[8:16, :], acc[24:32, :]], axis=0)
             + jnp.dot(wr_bridge, va, preferred_element_type=jnp.float32))
    for idx in range(8, _L - 1):
        b0 = b_ref[idx:idx + 1, 32 + idx:33 + idx]
        b1 = b_ref[idx:idx + 1, 48 + idx:49 + idx]
        v0 = jnp.maximum(acc_b[idx - 8:idx - 7, :] + b0, 0.0)
        v1 = jnp.maximum(acc_b[idx:idx + 1, :] + b1, 0.0)
        acc_b = (acc_b + wr_b[:, idx:idx + 1] * v0
                 + wr_b[:, 16 + idx:17 + idx] * v1)
        vs0.append(v0)
        vs1.append(v1)
    zero = jnp.zeros_like(vs0[0])
    a_t = jnp.concatenate(vs0 + [zero] + vs1 + [zero], axis=0)
    p15 = jnp.dot(xb, w15[:, :32].T,
                  preferred_element_type=jnp.float32)
    rec15 = jax.lax.dot_general(a_t, w15[:, 32:], _DN_TL,
                                preferred_element_type=jnp.float32)
    out_ref[...] = jnp.maximum(p15 + rec15 + b_ref[_L - 1:_L, :], 0.0)


def kernel(x, W, b):
    batch = x.shape[0]
    bb = _BB if batch % _BB == 0 else batch
    w2 = W.reshape(_L * 128, 64)
    grid = (batch // bb,)
    return pl.pallas_call(
        _prnn_block,
        grid=grid,
        in_specs=[
            pl.BlockSpec((bb, 128), lambda i: (i, 0)),
            pl.BlockSpec((_L * 128, 64), lambda i: (0, 0)),
            pl.BlockSpec((_L, 128), lambda i: (0, 0)),
        ],
        out_specs=pl.BlockSpec((bb, 128), lambda i: (i, 0)),
        out_shape=jax.ShapeDtypeStruct((batch, 128), jnp.float32),
        compiler_params=pltpu.CompilerParams(
            dimension_semantics=("parallel",)),
    )(x, w2, b)
